# baseline (speedup 1.0000x reference)
import functools

import jax
import jax.numpy as jnp
from jax import lax
from jax.experimental import pallas as pl
from jax.experimental.pallas import tpu as pltpu
from jax.experimental.pallas import tpu_sc as plsc

EPS = 1e-09
N = 10000
E = 640000
NW = 32
CH = 80
BM = 1024
BN = 1280


def _sc_gather(table, idx):
    e = idx.shape[0]
    per_w = e // NW
    nch = per_w // CH
    idx3 = idx.reshape(NW, nch, CH)
    mesh = plsc.VectorSubcoreMesh(core_axis_name="c", subcore_axis_name="s")

    @functools.partial(
        pl.kernel,
        out_type=jax.ShapeDtypeStruct((e,) + table.shape[1:], table.dtype),
        mesh=mesh,
        compiler_params=pltpu.CompilerParams(use_tc_tiling_on_sc=False),
        scratch_types=[
            pltpu.VMEM((nch, CH), jnp.int32),
            pltpu.VMEM((CH,) + table.shape[1:], table.dtype),
            pltpu.SemaphoreType.DMA,
        ],
    )
    def k(table_hbm, idx_hbm, out_hbm, idx_v, buf_v, sem):
        wid = lax.axis_index("s") * 2 + lax.axis_index("c")
        row0 = wid * nch
        pltpu.sync_copy(idx_hbm.at[wid], idx_v)

        def body(j, carry):
            pltpu.async_copy(table_hbm.at[idx_v.at[j]], buf_v, sem).wait()
            pltpu.sync_copy(buf_v, out_hbm.at[pl.ds((row0 + j) * CH, CH)])
            return carry

        lax.fori_loop(0, nch, body, 0)

    return k(table, idx3)


def _sc_scatter_add(vals, idx, n):
    e, d = vals.shape
    per_w = e // NW
    nch = per_w // CH
    idx3 = idx.reshape(NW, nch, CH)
    rpt = n // 16
    zeros = jnp.zeros((n, d), jnp.float32)
    mesh = plsc.VectorSubcoreMesh(core_axis_name="c", subcore_axis_name="s")

    @functools.partial(
        pl.kernel,
        out_type=jax.ShapeDtypeStruct((2, n, d), jnp.float32),
        mesh=mesh,
        compiler_params=pltpu.CompilerParams(use_tc_tiling_on_sc=False),
        scratch_types=[
            pltpu.VMEM((nch, CH), jnp.int32),
            pltpu.VMEM((CH, d), jnp.float32),
            pltpu.VMEM_SHARED((n, d), jnp.float32),
            pltpu.SemaphoreType.DMA,
        ],
    )
    def k(vals_hbm, idx_hbm, zeros_hbm, out_hbm, idx_v, buf_v, acc_sh, sem):
        cid = lax.axis_index("c")
        sid = lax.axis_index("s")
        wid = sid * 2 + cid
        pltpu.sync_copy(zeros_hbm.at[pl.ds(sid * rpt, rpt)],
                        acc_sh.at[pl.ds(sid * rpt, rpt)])
        plsc.subcore_barrier()
        pltpu.sync_copy(idx_hbm.at[wid], idx_v)

        def body(j, carry):
            pltpu.async_copy(
                vals_hbm.at[pl.ds((wid * nch + j) * CH, CH)], buf_v, sem).wait()
            pltpu.sync_copy(buf_v, acc_sh.at[idx_v.at[j]], add=True)
            return carry

        lax.fori_loop(0, nch, body, 0)
        plsc.subcore_barrier()
        pltpu.sync_copy(acc_sh.at[pl.ds(sid * rpt, rpt)],
                        out_hbm.at[cid].at[pl.ds(sid * rpt, rpt)])

    return k(vals, idx3, zeros)


BE = 2560


def _edge1_body(c_ref, ea_ref, emb_ref, w1_ref, b1_ref, nw_ref, o_ref):
    c = c_ref[0, 0, :].reshape(BE, 1)
    iota = jax.lax.broadcasted_iota(jnp.int32, (BE, 20), 1)
    oh = (c == iota).astype(jnp.float32)
    hsrc = jnp.dot(oh, emb_ref[...], preferred_element_type=jnp.float32)
    e1 = jnp.dot(ea_ref[...], w1_ref[...], preferred_element_type=jnp.float32)
    m = jax.nn.relu(hsrc + e1 + b1_ref[...])
    o_ref[...] = jnp.dot(m, nw_ref[...], preferred_element_type=jnp.float32)


def _edge1(c, ea, emb, w1, b1, nw):
    c3 = c.reshape(E // BE, 1, BE)
    return pl.pallas_call(
        _edge1_body,
        grid=(E // BE,),
        in_specs=[
            pl.BlockSpec((1, 1, BE), lambda i: (i, 0, 0)),
            pl.BlockSpec((BE, 16), lambda i: (i, 0)),
            pl.BlockSpec((20, 128), lambda i: (0, 0)),
            pl.BlockSpec((16, 128), lambda i: (0, 0)),
            pl.BlockSpec((1, 128), lambda i: (0, 0)),
            pl.BlockSpec((128, 64), lambda i: (0, 0)),
        ],
        out_specs=pl.BlockSpec((BE, 64), lambda i: (i, 0)),
        out_shape=jax.ShapeDtypeStruct((E, 64), jnp.float32),
    )(c3, ea, emb, w1, b1.reshape(1, 128), nw)


def _edge2_body(g_ref, ea_ref, w2_ref, b2_ref, nw_ref, o_ref):
    e2 = jnp.dot(ea_ref[...], w2_ref[...], preferred_element_type=jnp.float32)
    m = jax.nn.relu(g_ref[...] + e2 + b2_ref[...])
    o_ref[...] = jnp.dot(m, nw_ref[...], preferred_element_type=jnp.float32)


def _edge2(g, ea, w2, b2, nw):
    return pl.pallas_call(
        _edge2_body,
        grid=(E // BE,),
        in_specs=[
            pl.BlockSpec((BE, 64), lambda i: (i, 0)),
            pl.BlockSpec((BE, 16), lambda i: (i, 0)),
            pl.BlockSpec((16, 64), lambda i: (0, 0)),
            pl.BlockSpec((1, 64), lambda i: (0, 0)),
            pl.BlockSpec((64, 32), lambda i: (0, 0)),
        ],
        out_specs=pl.BlockSpec((BE, 32), lambda i: (i, 0)),
        out_shape=jax.ShapeDtypeStruct((E, 32), jnp.float32),
    )(g, ea, w2, b2.reshape(1, 64), nw)


def _gram_body(a_ref, b_ref, o_ref):
    o_ref[...] = jax.lax.dot_general(
        a_ref[...], b_ref[...], (((1,), (1,)), ((), ())),
        preferred_element_type=jnp.float32)


def _gram(h):
    n = h.shape[0]
    grid = (pl.cdiv(n, BM), pl.cdiv(n, BN))
    return pl.pallas_call(
        _gram_body,
        grid=grid,
        in_specs=[
            pl.BlockSpec((BM, h.shape[1]), lambda i, j: (i, 0)),
            pl.BlockSpec((BN, h.shape[1]), lambda i, j: (j, 0)),
        ],
        out_specs=pl.BlockSpec((BM, BN), lambda i, j: (i, j)),
        out_shape=jax.ShapeDtypeStruct((n, n), jnp.float32),
    )(h, h)


def kernel(x, edge_index, edge_attr, emb, lin_edge1_w, lin_edge1_b, nn1_w, nn1_b,
           lin_edge2_w, lin_edge2_b, nn2_w, nn2_b):
    src, dst = edge_index[0], edge_index[1]

    c = _sc_gather(jnp.broadcast_to(x, (N, 16)), src)[:, 0]

    h = jnp.take(emb, x[:, 0], axis=0)
    p1 = _edge1(c, edge_attr, emb, lin_edge1_w, lin_edge1_b, nn1_w)
    part = _sc_scatter_add(p1, dst, N)
    h = part[0] + part[1] + (1.0 + EPS) * (h @ nn1_w) + nn1_b
    h = jax.nn.leaky_relu(h, negative_slope=0.01)

    g = _sc_gather(h, src)
    p2 = _edge2(g, edge_attr, lin_edge2_w, lin_edge2_b, nn2_w)
    part = _sc_scatter_add(p2, dst, N)
    h = part[0] + part[1] + (1.0 + EPS) * (h @ nn2_w) + nn2_b

    return _gram(h)


# speedup vs baseline: 1.2385x; 1.2385x over previous
import functools

import jax
import jax.numpy as jnp
from jax import lax
from jax.experimental import pallas as pl
from jax.experimental.pallas import tpu as pltpu
from jax.experimental.pallas import tpu_sc as plsc

EPS = 1e-09
N = 10000
E = 640000
NW = 32
CH = 80
NBUF = 5
LOOK = 3
BM = 1024
BN = 1280


def _sc_gather(table, idx):
    e = idx.shape[0]
    per_w = e // NW
    nch = per_w // CH
    idx3 = idx.reshape(NW, nch, CH)
    mesh = plsc.VectorSubcoreMesh(core_axis_name="c", subcore_axis_name="s")

    @functools.partial(
        pl.kernel,
        out_type=jax.ShapeDtypeStruct((e,) + table.shape[1:], table.dtype),
        mesh=mesh,
        compiler_params=pltpu.CompilerParams(use_tc_tiling_on_sc=False),
        scratch_types=[
            pltpu.VMEM((nch, CH), jnp.int32),
            pltpu.VMEM((NBUF, CH) + table.shape[1:], table.dtype),
            pltpu.SemaphoreType.DMA((NBUF,)),
            pltpu.SemaphoreType.DMA((NBUF,)),
        ],
    )
    def k(table_hbm, idx_hbm, out_hbm, idx_v, bufs_v, gsem, ssem):
        wid = lax.axis_index("s") * 2 + lax.axis_index("c")
        row0 = wid * nch
        pltpu.sync_copy(idx_hbm.at[wid], idx_v)

        for j in range(LOOK):
            pltpu.async_copy(table_hbm.at[idx_v.at[j]], bufs_v.at[j], gsem.at[j])

        def group(g, carry):
            for b in range(NBUF):
                j = g * NBUF + b
                pltpu.make_async_copy(
                    table_hbm.at[idx_v.at[0]], bufs_v.at[b], gsem.at[b]).wait()
                pltpu.async_copy(
                    bufs_v.at[b], out_hbm.at[pl.ds((row0 + j) * CH, CH)],
                    ssem.at[b])
                jn = j + LOOK
                bn = (b + LOOK) % NBUF

                @pl.when(jn < nch)
                def _():
                    @pl.when(jn >= NBUF)
                    def _():
                        pltpu.make_async_copy(
                            bufs_v.at[bn], out_hbm.at[pl.ds(0, CH)],
                            ssem.at[bn]).wait()
                    pltpu.async_copy(
                        table_hbm.at[idx_v.at[jn]], bufs_v.at[bn], gsem.at[bn])
            return carry

        lax.fori_loop(0, nch // NBUF, group, 0)
        for b in range(NBUF):
            pltpu.make_async_copy(
                bufs_v.at[b], out_hbm.at[pl.ds(0, CH)], ssem.at[b]).wait()

    return k(table, idx3)


def _sc_scatter_add(vals, idx, n):
    e, d = vals.shape
    per_w = e // NW
    nch = per_w // CH
    idx3 = idx.reshape(NW, nch, CH)
    rpt = n // 16
    zeros = jnp.zeros((n, d), jnp.float32)
    mesh = plsc.VectorSubcoreMesh(core_axis_name="c", subcore_axis_name="s")

    @functools.partial(
        pl.kernel,
        out_type=jax.ShapeDtypeStruct((2, n, d), jnp.float32),
        mesh=mesh,
        compiler_params=pltpu.CompilerParams(use_tc_tiling_on_sc=False),
        scratch_types=[
            pltpu.VMEM((nch, CH), jnp.int32),
            pltpu.VMEM((NBUF, CH, d), jnp.float32),
            pltpu.VMEM_SHARED((n, d), jnp.float32),
            pltpu.SemaphoreType.DMA((NBUF,)),
            pltpu.SemaphoreType.DMA((NBUF,)),
        ],
    )
    def k(vals_hbm, idx_hbm, zeros_hbm, out_hbm, idx_v, bufs_v, acc_sh, gsem, ssem):
        cid = lax.axis_index("c")
        sid = lax.axis_index("s")
        wid = sid * 2 + cid
        row0 = wid * nch
        pltpu.sync_copy(zeros_hbm.at[pl.ds(sid * rpt, rpt)],
                        acc_sh.at[pl.ds(sid * rpt, rpt)])
        plsc.subcore_barrier()
        pltpu.sync_copy(idx_hbm.at[wid], idx_v)

        for j in range(LOOK):
            pltpu.async_copy(
                vals_hbm.at[pl.ds((row0 + j) * CH, CH)], bufs_v.at[j],
                gsem.at[j])

        def group(g, carry):
            for b in range(NBUF):
                j = g * NBUF + b
                pltpu.make_async_copy(
                    vals_hbm.at[pl.ds(0, CH)], bufs_v.at[b], gsem.at[b]).wait()
                pltpu.async_copy(
                    bufs_v.at[b], acc_sh.at[idx_v.at[j]], ssem.at[b], add=True)
                jn = j + LOOK
                bn = (b + LOOK) % NBUF

                @pl.when(jn < nch)
                def _():
                    @pl.when(jn >= NBUF)
                    def _():
                        pltpu.make_async_copy(
                            bufs_v.at[bn], acc_sh.at[idx_v.at[0]],
                            ssem.at[bn]).wait()
                    pltpu.async_copy(
                        vals_hbm.at[pl.ds((row0 + jn) * CH, CH)], bufs_v.at[bn],
                        gsem.at[bn])
            return carry

        lax.fori_loop(0, nch // NBUF, group, 0)
        for b in range(NBUF):
            pltpu.make_async_copy(
                bufs_v.at[b], acc_sh.at[idx_v.at[0]], ssem.at[b]).wait()
        plsc.subcore_barrier()
        pltpu.sync_copy(acc_sh.at[pl.ds(sid * rpt, rpt)],
                        out_hbm.at[cid].at[pl.ds(sid * rpt, rpt)])

    return k(vals, idx3, zeros)


BE = 2560


def _edge1_body(c_ref, ea_ref, emb_ref, w1_ref, b1_ref, nw_ref, o_ref):
    c = c_ref[0, 0, :].reshape(BE, 1)
    iota = jax.lax.broadcasted_iota(jnp.int32, (BE, 20), 1)
    oh = (c == iota).astype(jnp.float32)
    hsrc = jnp.dot(oh, emb_ref[...], preferred_element_type=jnp.float32)
    e1 = jnp.dot(ea_ref[...], w1_ref[...], preferred_element_type=jnp.float32)
    m = jax.nn.relu(hsrc + e1 + b1_ref[...])
    o_ref[...] = jnp.dot(m, nw_ref[...], preferred_element_type=jnp.float32)


def _edge1(c, ea, emb, w1, b1, nw):
    c3 = c.reshape(E // BE, 1, BE)
    return pl.pallas_call(
        _edge1_body,
        grid=(E // BE,),
        in_specs=[
            pl.BlockSpec((1, 1, BE), lambda i: (i, 0, 0)),
            pl.BlockSpec((BE, 16), lambda i: (i, 0)),
            pl.BlockSpec((20, 128), lambda i: (0, 0)),
            pl.BlockSpec((16, 128), lambda i: (0, 0)),
            pl.BlockSpec((1, 128), lambda i: (0, 0)),
            pl.BlockSpec((128, 64), lambda i: (0, 0)),
        ],
        out_specs=pl.BlockSpec((BE, 64), lambda i: (i, 0)),
        out_shape=jax.ShapeDtypeStruct((E, 64), jnp.float32),
    )(c3, ea, emb, w1, b1.reshape(1, 128), nw)


def _edge2_body(g_ref, ea_ref, w2_ref, b2_ref, nw_ref, o_ref):
    e2 = jnp.dot(ea_ref[...], w2_ref[...], preferred_element_type=jnp.float32)
    m = jax.nn.relu(g_ref[...] + e2 + b2_ref[...])
    o_ref[...] = jnp.dot(m, nw_ref[...], preferred_element_type=jnp.float32)


def _edge2(g, ea, w2, b2, nw):
    return pl.pallas_call(
        _edge2_body,
        grid=(E // BE,),
        in_specs=[
            pl.BlockSpec((BE, 64), lambda i: (i, 0)),
            pl.BlockSpec((BE, 16), lambda i: (i, 0)),
            pl.BlockSpec((16, 64), lambda i: (0, 0)),
            pl.BlockSpec((1, 64), lambda i: (0, 0)),
            pl.BlockSpec((64, 32), lambda i: (0, 0)),
        ],
        out_specs=pl.BlockSpec((BE, 32), lambda i: (i, 0)),
        out_shape=jax.ShapeDtypeStruct((E, 32), jnp.float32),
    )(g, ea, w2, b2.reshape(1, 64), nw)


def _gram_body(a_ref, b_ref, o_ref):
    o_ref[...] = jax.lax.dot_general(
        a_ref[...], b_ref[...], (((1,), (1,)), ((), ())),
        preferred_element_type=jnp.float32)


def _gram(h):
    n = h.shape[0]
    grid = (pl.cdiv(n, BM), pl.cdiv(n, BN))
    return pl.pallas_call(
        _gram_body,
        grid=grid,
        in_specs=[
            pl.BlockSpec((BM, h.shape[1]), lambda i, j: (i, 0)),
            pl.BlockSpec((BN, h.shape[1]), lambda i, j: (j, 0)),
        ],
        out_specs=pl.BlockSpec((BM, BN), lambda i, j: (i, j)),
        out_shape=jax.ShapeDtypeStruct((n, n), jnp.float32),
    )(h, h)


def kernel(x, edge_index, edge_attr, emb, lin_edge1_w, lin_edge1_b, nn1_w, nn1_b,
           lin_edge2_w, lin_edge2_b, nn2_w, nn2_b):
    src, dst = edge_index[0], edge_index[1]

    c = _sc_gather(jnp.broadcast_to(x, (N, 16)), src)[:, 0]

    h = jnp.take(emb, x[:, 0], axis=0)
    p1 = _edge1(c, edge_attr, emb, lin_edge1_w, lin_edge1_b, nn1_w)
    part = _sc_scatter_add(p1, dst, N)
    h = part[0] + part[1] + (1.0 + EPS) * (h @ nn1_w) + nn1_b
    h = jax.nn.leaky_relu(h, negative_slope=0.01)

    g = _sc_gather(h, src)
    p2 = _edge2(g, edge_attr, lin_edge2_w, lin_edge2_b, nn2_w)
    part = _sc_scatter_add(p2, dst, N)
    h = part[0] + part[1] + (1.0 + EPS) * (h @ nn2_w) + nn2_b

    return _gram(h)
